# baseline (speedup 1.0000x reference)
import functools

import jax
import jax.numpy as jnp
from jax import lax
from jax.experimental import pallas as pl
from jax.experimental.pallas import tpu as pltpu
from jax.experimental.pallas import tpu_sc as plsc

VOCAB = 1000000
EMBED = 32
BATCH = 4096
SEQ = 200

NC = 2
NS = 16
NW = NC * NS
B_TOTAL = BATCH * SEQ
ROWS_PW = B_TOTAL // NW
B_PW = BATCH // NW
NB = 4
RPB = NB * SEQ
IPS = SEQ
K = RPB // IPS
NBLK = B_PW // NB


@functools.partial(
    pl.kernel,
    out_type=jax.ShapeDtypeStruct((BATCH, SEQ, EMBED), jnp.float32),
    mesh=plsc.VectorSubcoreMesh(core_axis_name="c", subcore_axis_name="s"),
    scratch_types=[
        pltpu.VMEM((ROWS_PW,), jnp.int32),
        pltpu.VMEM((NB, SEQ, EMBED), jnp.float32),
        pltpu.SemaphoreType.DMA,
    ],
    compiler_params=pltpu.CompilerParams(use_tc_tiling_on_sc=False),
)
def _emb_lookup(x_hbm, table_hbm, out_hbm, idx_v, rows_v, gsem):
    wid = lax.axis_index("s") * NC + lax.axis_index("c")
    pltpu.sync_copy(x_hbm.at[wid], idx_v)
    b_base = wid * B_PW

    def blk_body(blk, carry):
        descs = [
            pltpu.async_copy(
                table_hbm.at[idx_v.at[pl.ds(blk * RPB + j * IPS, IPS)]],
                rows_v.at[j],
                gsem,
            )
            for j in range(K)
        ]
        for d in descs:
            d.wait()
        pltpu.sync_copy(rows_v, out_hbm.at[pl.ds(b_base + blk * NB, NB)])
        return carry

    lax.fori_loop(0, NBLK, blk_body, 0)


def kernel(x, table):
    x_r = x.reshape(NW, ROWS_PW).astype(jnp.int32)
    return _emb_lookup(x_r, table)
